# Initial kernel scaffold; baseline (speedup 1.0000x reference)
import jax
import jax.numpy as jnp
from jax.experimental import pallas as pl


def kernel(outputs, inputs, enc1, dec1, masks, segs, confidence, iteration, epoch):
    raise NotImplementedError("write your pallas kernel here")


# speedup vs baseline: 2.5006x; 2.5006x over previous
import functools

import jax
import jax.numpy as jnp
from jax import lax
from jax.experimental import pallas as pl
from jax.experimental.pallas import tpu as pltpu
from jax.experimental.pallas import tpu_sc as plsc

_B, _C, _H, _W = 8, 4, 512, 512
_CE, _HE, _WE = 128, 128, 128
_NSEG = 64
_NPIX = _HE * _WE
_KCH = 8
_HB = _H // _KCH
_HEB = _HE // _KCH
_NW = 32
_PPW = _NPIX * _B // _NW
_RPW = _PPW // 16


def _dense_body(o_ref, i_ref, m_ref, e_ref, d_ref, err_ref, sums_ref, acc_ref):
    b = pl.program_id(0)
    k = pl.program_id(1)

    @pl.when((b == 0) & (k == 0))
    def _init():
        acc_ref[0] = 0.0
        acc_ref[1] = 0.0

    m = m_ref[0, 0]
    o = o_ref[0]
    x = i_ref[0]
    t = jnp.where(m[None] >= 0.5, 0.0, x)
    dd = o - t
    mse = jnp.sum(dd * dd, axis=0)
    w = (m > 0.0).astype(jnp.float32)
    acc_ref[0] += jnp.sum(mse * w)
    acc_ref[1] += jnp.sum(w)

    de = e_ref[0] - d_ref[0]
    err_ref[0] = jnp.sum(de * de, axis=0) * (1.0 / _CE)

    @pl.when((b == _B - 1) & (k == _KCH - 1))
    def _fini():
        sums_ref[0] = acc_ref[0]
        sums_ref[1] = acc_ref[1]


def _dense_pass(outputs, inputs, masks, enc1, dec1):
    return pl.pallas_call(
        _dense_body,
        grid=(_B, _KCH),
        in_specs=[
            pl.BlockSpec((1, _C, _HB, _W), lambda b, k: (b, 0, k, 0)),
            pl.BlockSpec((1, _C, _HB, _W), lambda b, k: (b, 0, k, 0)),
            pl.BlockSpec((1, 1, _HB, _W), lambda b, k: (b, 0, k, 0)),
            pl.BlockSpec((1, _CE, _HEB, _WE), lambda b, k: (b, 0, k, 0)),
            pl.BlockSpec((1, _CE, _HEB, _WE), lambda b, k: (b, 0, k, 0)),
        ],
        out_specs=[
            pl.BlockSpec((1, _HEB, _WE), lambda b, k: (b, k, 0)),
            pl.BlockSpec(memory_space=pltpu.SMEM),
        ],
        out_shape=[
            jax.ShapeDtypeStruct((_B, _HE, _WE), jnp.float32),
            jax.ShapeDtypeStruct((2,), jnp.float32),
        ],
        scratch_shapes=[pltpu.SMEM((2,), jnp.float32)],
    )(outputs, inputs, masks, enc1, dec1)


def _sc_body(seg_hbm, err_hbm, mask_hbm, out_hbm, seg_v, err_v, mask_v, table):
    c = lax.axis_index("c")
    s = lax.axis_index("s")
    wid = s * 2 + c
    row0 = wid * _RPW

    pltpu.sync_copy(seg_hbm.at[pl.ds(row0, _RPW)], seg_v)
    pltpu.sync_copy(err_hbm.at[pl.ds(row0, _RPW)], err_v)
    pltpu.sync_copy(mask_hbm.at[pl.ds(row0, _RPW)], mask_v)

    zf = jnp.zeros((16,), jnp.float32)
    for r in range(3 * _NSEG):
        table[pl.ds(r * 16, 16)] = zf

    lane = lax.iota(jnp.int32, 16)
    ones_f = jnp.full((16,), 1.0, jnp.float32)

    def body(i, carry):
        sg = seg_v[i]
        e = err_v[i]
        m = mask_v[i]
        pos = jnp.where((m > 0.0) & (m < 0.5), 1.0, 0.0)
        base = sg * 16 + lane
        plsc.addupdate_scatter(table, [base], ones_f)
        plsc.addupdate_scatter(table, [base + (_NSEG * 16)], e)
        plsc.addupdate_scatter(table, [base + (2 * _NSEG * 16)], pos)
        return carry

    lax.fori_loop(0, _RPW, body, 0)

    pltpu.sync_copy(table, out_hbm.at[wid])


def _sc_segsum(seg2d, err2d, mask2d):
    mesh = plsc.VectorSubcoreMesh(core_axis_name="c", subcore_axis_name="s")
    fn = functools.partial(
        pl.kernel,
        mesh=mesh,
        compiler_params=pltpu.CompilerParams(needs_layout_passes=False),
        out_type=jax.ShapeDtypeStruct((_NW, 3 * _NSEG * 16), jnp.float32),
        scratch_types=[
            pltpu.VMEM((_RPW, 16), jnp.int32),
            pltpu.VMEM((_RPW, 16), jnp.float32),
            pltpu.VMEM((_RPW, 16), jnp.float32),
            pltpu.VMEM((3 * _NSEG * 16,), jnp.float32),
        ],
    )(_sc_body)
    return fn(seg2d, err2d, mask2d)


def _epi_body(p_ref, s_ref, o_ref):
    t = jnp.sum(p_ref[...], axis=3)
    num = 0.0
    den = 0.0
    for b in range(_B):
        g = t[4 * b] + t[4 * b + 1] + t[4 * b + 2] + t[4 * b + 3]
        counts = g[0]
        errs = g[1]
        pos = g[2]
        cm = jnp.maximum(counts, 1.0)
        mean_err = errs / cm
        valid = (counts / _NPIX) >= 0.01
        is_pos = (pos / cm) > 0.01
        sel = jnp.where(valid & is_pos, 1.0, 0.0)
        num += jnp.sum(mean_err * sel)
        den += jnp.sum(sel)
    o_ref[0] = s_ref[0] / jnp.maximum(s_ref[1], 1.0) + num / jnp.maximum(den, 1.0)


def _epilogue(partials, sums):
    return pl.pallas_call(
        _epi_body,
        in_specs=[
            pl.BlockSpec(memory_space=pltpu.VMEM),
            pl.BlockSpec(memory_space=pltpu.SMEM),
        ],
        out_specs=pl.BlockSpec(memory_space=pltpu.SMEM),
        out_shape=jax.ShapeDtypeStruct((1,), jnp.float32),
    )(partials, sums)


def kernel(outputs, inputs, enc1, dec1, masks, segs, confidence, iteration, epoch):
    seg_ds = segs[:, 0, ::4, ::4].astype(jnp.int32).reshape(_B * _NPIX // 16, 16)
    mask_ds = masks[:, 0, ::4, ::4].reshape(_B * _NPIX // 16, 16)

    err, sums = _dense_pass(outputs, inputs, masks, enc1, dec1)
    partials = _sc_segsum(seg_ds, err.reshape(_B * _NPIX // 16, 16), mask_ds)
    loss = _epilogue(partials.reshape(_NW, 3, _NSEG, 16), sums)
    return loss[0]


# speedup vs baseline: 2.7277x; 1.0908x over previous
import functools

import jax
import jax.numpy as jnp
from jax import lax
from jax.experimental import pallas as pl
from jax.experimental.pallas import tpu as pltpu
from jax.experimental.pallas import tpu_sc as plsc

_B, _C, _H, _W = 8, 4, 512, 512
_CE, _HE, _WE = 128, 128, 128
_NSEG = 64
_NPIX = _HE * _WE
_KCH = 4
_HB = _H // _KCH
_HEB = _HE // _KCH
_NW = 32
_PPW = _NPIX * _B // _NW
_RPW = _PPW // 16


def _dense_body(o_ref, i_ref, m_ref, e_ref, d_ref, err_ref, sums_ref, acc_ref):
    b = pl.program_id(0)
    k = pl.program_id(1)

    @pl.when((b == 0) & (k == 0))
    def _init():
        acc_ref[0] = 0.0
        acc_ref[1] = 0.0

    m = m_ref[0, 0]
    o = o_ref[0]
    x = i_ref[0]
    t = jnp.where(m[None] >= 0.5, 0.0, x)
    dd = o - t
    mse = jnp.sum(dd * dd, axis=0)
    w = (m > 0.0).astype(jnp.float32)
    acc_ref[0] += jnp.sum(mse * w)
    acc_ref[1] += jnp.sum(w)

    de = e_ref[0] - d_ref[0]
    err_ref[0] = jnp.sum(de * de, axis=0) * (1.0 / _CE)

    @pl.when((b == _B - 1) & (k == _KCH - 1))
    def _fini():
        sums_ref[0] = acc_ref[0]
        sums_ref[1] = acc_ref[1]


def _dense_pass(outputs, inputs, masks, enc1, dec1):
    return pl.pallas_call(
        _dense_body,
        grid=(_B, _KCH),
        in_specs=[
            pl.BlockSpec((1, _C, _HB, _W), lambda b, k: (b, 0, k, 0)),
            pl.BlockSpec((1, _C, _HB, _W), lambda b, k: (b, 0, k, 0)),
            pl.BlockSpec((1, 1, _HB, _W), lambda b, k: (b, 0, k, 0)),
            pl.BlockSpec((1, _CE, _HEB, _WE), lambda b, k: (b, 0, k, 0)),
            pl.BlockSpec((1, _CE, _HEB, _WE), lambda b, k: (b, 0, k, 0)),
        ],
        out_specs=[
            pl.BlockSpec((1, _HEB, _WE), lambda b, k: (b, k, 0)),
            pl.BlockSpec(memory_space=pltpu.SMEM),
        ],
        out_shape=[
            jax.ShapeDtypeStruct((_B, _HE, _WE), jnp.float32),
            jax.ShapeDtypeStruct((2,), jnp.float32),
        ],
        scratch_shapes=[pltpu.SMEM((2,), jnp.float32)],
    )(outputs, inputs, masks, enc1, dec1)


def _sc_body(seg_hbm, err_hbm, mask_hbm, out_hbm, seg_v, err_v, mask_v, table):
    c = lax.axis_index("c")
    s = lax.axis_index("s")
    wid = s * 2 + c
    row0 = wid * _RPW

    pltpu.sync_copy(seg_hbm.at[pl.ds(row0, _RPW)], seg_v)
    pltpu.sync_copy(err_hbm.at[pl.ds(row0, _RPW)], err_v)
    pltpu.sync_copy(mask_hbm.at[pl.ds(row0, _RPW)], mask_v)

    zf = jnp.zeros((16,), jnp.float32)
    for r in range(3 * _NSEG):
        table[pl.ds(r * 16, 16)] = zf

    lane = lax.iota(jnp.int32, 16)
    ones_f = jnp.full((16,), 1.0, jnp.float32)

    def body(i, carry):
        sg = seg_v[i]
        e = err_v[i]
        m = mask_v[i]
        pos = jnp.where((m > 0.0) & (m < 0.5), 1.0, 0.0)
        base = sg * 16 + lane
        plsc.addupdate_scatter(table, [base], ones_f)
        plsc.addupdate_scatter(table, [base + (_NSEG * 16)], e)
        plsc.addupdate_scatter(table, [base + (2 * _NSEG * 16)], pos)
        return carry

    lax.fori_loop(0, _RPW, body, 0)

    pltpu.sync_copy(table, out_hbm.at[wid])


def _sc_segsum(seg2d, err2d, mask2d):
    mesh = plsc.VectorSubcoreMesh(core_axis_name="c", subcore_axis_name="s")
    fn = functools.partial(
        pl.kernel,
        mesh=mesh,
        compiler_params=pltpu.CompilerParams(needs_layout_passes=False),
        out_type=jax.ShapeDtypeStruct((_NW, 3 * _NSEG * 16), jnp.float32),
        scratch_types=[
            pltpu.VMEM((_RPW, 16), jnp.int32),
            pltpu.VMEM((_RPW, 16), jnp.float32),
            pltpu.VMEM((_RPW, 16), jnp.float32),
            pltpu.VMEM((3 * _NSEG * 16,), jnp.float32),
        ],
    )(_sc_body)
    return fn(seg2d, err2d, mask2d)


def _epi_body(p_ref, s_ref, o_ref):
    t = jnp.sum(p_ref[...], axis=3)
    num = 0.0
    den = 0.0
    for b in range(_B):
        g = t[4 * b] + t[4 * b + 1] + t[4 * b + 2] + t[4 * b + 3]
        counts = g[0]
        errs = g[1]
        pos = g[2]
        cm = jnp.maximum(counts, 1.0)
        mean_err = errs / cm
        valid = (counts / _NPIX) >= 0.01
        is_pos = (pos / cm) > 0.01
        sel = jnp.where(valid & is_pos, 1.0, 0.0)
        num += jnp.sum(mean_err * sel)
        den += jnp.sum(sel)
    o_ref[0] = s_ref[0] / jnp.maximum(s_ref[1], 1.0) + num / jnp.maximum(den, 1.0)


def _epilogue(partials, sums):
    return pl.pallas_call(
        _epi_body,
        in_specs=[
            pl.BlockSpec(memory_space=pltpu.VMEM),
            pl.BlockSpec(memory_space=pltpu.SMEM),
        ],
        out_specs=pl.BlockSpec(memory_space=pltpu.SMEM),
        out_shape=jax.ShapeDtypeStruct((1,), jnp.float32),
    )(partials, sums)


def kernel(outputs, inputs, enc1, dec1, masks, segs, confidence, iteration, epoch):
    seg_ds = segs[:, 0, ::4, ::4].astype(jnp.int32).reshape(_B * _NPIX // 16, 16)
    mask_ds = masks[:, 0, ::4, ::4].reshape(_B * _NPIX // 16, 16)

    err, sums = _dense_pass(outputs, inputs, masks, enc1, dec1)
    partials = _sc_segsum(seg_ds, err.reshape(_B * _NPIX // 16, 16), mask_ds)
    loss = _epilogue(partials.reshape(_NW, 3, _NSEG, 16), sums)
    return loss[0]
